# baseline (speedup 1.0000x reference)
import functools

import jax
import jax.numpy as jnp
from jax import lax
from jax.experimental import pallas as pl
from jax.experimental.pallas import tpu as pltpu
from jax.experimental.pallas import tpu_sc as plsc

_CB = 8192
_D = 32
_TM = 256
_KC = 2048
_NKC = _CB // _KC


def _c2_body(ct_ref, c2_ref):
    c = ct_ref[...]
    c2_ref[...] = jnp.sum(c * c, axis=0, keepdims=True)


def _sq_norms(ct):
    return pl.pallas_call(
        _c2_body,
        grid=(1,),
        in_specs=[pl.BlockSpec((_D, _CB), lambda i: (0, 0))],
        out_specs=pl.BlockSpec((1, _CB), lambda i: (0, 0)),
        out_shape=jax.ShapeDtypeStruct((1, _CB), jnp.float32),
    )(ct)


def _vq_argmin_body(zt_ref, ct_ref, c2_ref, tok_ref, loss_ref):
    zt = zt_ref[...]
    z2 = jnp.sum(zt * zt, axis=0, keepdims=True)
    norm = jnp.sqrt(z2)
    mnorm = jnp.maximum(norm, 1e-12)
    znt = zt / mnorm
    x2 = jnp.sum(znt * znt, axis=0, keepdims=True)
    znt_bf = znt.astype(jnp.bfloat16)

    def body(kk, carry):
        run_min, run_idx, run_zc, run_c2 = carry
        ctb = ct_ref[:, pl.ds(kk * _KC, _KC)]
        c2b = c2_ref[pl.ds(kk * _KC, _KC), :]
        zct = lax.dot_general(ctb.astype(jnp.bfloat16), znt_bf,
                              (((0,), (0,)), ((), ())),
                              preferred_element_type=jnp.float32)
        d2 = x2 + c2b - 2.0 * zct
        dd = jnp.sqrt(jnp.clip(d2, 0.0, None))
        minv = jnp.min(dd, axis=0, keepdims=True)
        iota = lax.broadcasted_iota(jnp.int32, (_KC, _TM), 0) + kk * _KC
        idx = jnp.min(jnp.where(dd == minv, iota, jnp.int32(2**30)),
                      axis=0, keepdims=True)
        msk = iota == idx
        zcs = jnp.sum(jnp.where(msk, zct, 0.0), axis=0, keepdims=True)
        c2s = jnp.sum(jnp.where(msk, jnp.broadcast_to(c2b, (_KC, _TM)), 0.0),
                      axis=0, keepdims=True)
        upd = minv < run_min
        new_min = jnp.where(upd, minv, run_min)
        rounded = new_min.astype(jnp.bfloat16).astype(jnp.float32)
        new_min = jnp.where((kk % 2) == 1, rounded, new_min)
        return (new_min,
                jnp.where(upd, idx, run_idx),
                jnp.where(upd, zcs, run_zc),
                jnp.where(upd, c2s, run_c2))

    init = (jnp.full((1, _TM), jnp.inf, jnp.float32),
            jnp.zeros((1, _TM), jnp.int32),
            jnp.zeros((1, _TM), jnp.float32),
            jnp.zeros((1, _TM), jnp.float32))
    _, run_idx, run_zc, run_c2 = lax.fori_loop(0, _NKC, body, init)
    tok_ref[...] = run_idx.reshape(1, 1, _TM)
    part = jnp.sum(z2 - 2.0 * (mnorm * run_zc) + run_c2)

    @pl.when(pl.program_id(0) == 0)
    def _():
        loss_ref[0, 0] = part

    @pl.when(pl.program_id(0) != 0)
    def _():
        loss_ref[0, 0] += part


def _vq_argmin(zt, ct, c2col):
    n_rows = zt.shape[1]
    grid = n_rows // _TM
    return pl.pallas_call(
        _vq_argmin_body,
        grid=(grid,),
        in_specs=[
            pl.BlockSpec((_D, _TM), lambda i: (0, i)),
            pl.BlockSpec((_D, _CB), lambda i: (0, 0)),
            pl.BlockSpec((_CB, 1), lambda i: (0, 0)),
        ],
        out_specs=[
            pl.BlockSpec((1, 1, _TM), lambda i: (i, 0, 0)),
            pl.BlockSpec(memory_space=pltpu.SMEM, block_shape=(1, 1),
                         index_map=lambda i: (0, 0)),
        ],
        out_shape=[
            jax.ShapeDtypeStruct((grid, 1, _TM), jnp.int32),
            jax.ShapeDtypeStruct((1, 1), jnp.float32),
        ],
    )(zt, ct, c2col)


_DP = 128


def _make_sc_gather(n_rows):
    info = plsc.get_sparse_core_info()
    nc, ns = info.num_cores, info.num_subcores
    nw = nc * ns
    b_per_w = n_rows // nw
    mesh = plsc.VectorSubcoreMesh(core_axis_name="c", subcore_axis_name="s")

    @functools.partial(
        pl.kernel,
        out_type=jax.ShapeDtypeStruct((n_rows, _DP), jnp.float32),
        mesh=mesh,
        scratch_types=[
            pltpu.VMEM((b_per_w,), jnp.int32),
            pltpu.VMEM((b_per_w, _DP), jnp.float32),
            pltpu.SemaphoreType.DMA,
        ],
    )
    def gather_kernel(table_hbm, idx_hbm, out_hbm, idx_v, rows_v, sem):
        wid = lax.axis_index("s") * nc + lax.axis_index("c")
        base = wid * b_per_w
        pltpu.sync_copy(idx_hbm.at[pl.ds(base, b_per_w)], idx_v)
        pltpu.async_copy(table_hbm.at[idx_v], rows_v, sem).wait()
        pltpu.sync_copy(rows_v, out_hbm.at[pl.ds(base, b_per_w)])

    return gather_kernel


def kernel(z, codebook):
    B, N, D = z.shape
    n_rows = B * N
    zt = z.reshape(n_rows, D).T
    ct = codebook.T

    c2 = _sq_norms(ct)
    tok3, loss_sum = _vq_argmin(zt, ct, c2.reshape(_CB, 1))
    tokens_flat = tok3.reshape(-1)

    cb_pad = jnp.pad(codebook, ((0, 0), (0, _DP - D)))
    q = _make_sc_gather(n_rows)(cb_pad, tokens_flat)[:, :D]

    quantized = q.reshape(B, N, D)[:, None, :, :]
    tokens = tokens_flat.reshape(B, N)

    m = loss_sum[0, 0] / jnp.float32(n_rows * D)
    commitment_loss = jnp.float32(0.25) * m
    codebook_loss = m
    quantizer_loss = commitment_loss + codebook_loss
    return quantized, quantizer_loss, commitment_loss, codebook_loss, tokens
